# baseline (speedup 1.0000x reference)
import functools

import jax
import jax.numpy as jnp
from jax import lax
from jax.experimental import pallas as pl
from jax.experimental.pallas import tpu as pltpu
from jax.experimental.pallas import tpu_sc as plsc

VOCAB = 1000000
EMBED = 32
BATCH = 4096
SEQ = 200

NC = 2
NS = 16
NW = NC * NS
LT = SEQ // 8
HALF = 2
N_STEPS = SEQ // HALF
ROWS = HALF * 128


@functools.partial(
    pl.kernel,
    out_type=jax.ShapeDtypeStruct((SEQ, EMBED, BATCH), jnp.float32),
    mesh=plsc.VectorSubcoreMesh(
        core_axis_name="c", subcore_axis_name="s",
        num_cores=NC, num_subcores=NS),
    scratch_types=[
        pltpu.VMEM((ROWS,), jnp.int32),
        pltpu.VMEM((ROWS,), jnp.int32),
        pltpu.VMEM((ROWS,), jnp.int32),
        pltpu.VMEM((ROWS,), jnp.int32),
        pltpu.VMEM((ROWS, 128), jnp.float32),
        pltpu.VMEM((ROWS, 128), jnp.float32),
        pltpu.VMEM((HALF, EMBED, 128), jnp.float32),
        pltpu.VMEM((HALF, EMBED, 128), jnp.float32),
        pltpu.SemaphoreType.DMA,
        pltpu.SemaphoreType.DMA,
        pltpu.SemaphoreType.DMA,
        pltpu.SemaphoreType.DMA,
        pltpu.SemaphoreType.DMA,
        pltpu.SemaphoreType.DMA,
    ],
    compiler_params=pltpu.CompilerParams(use_tc_tiling_on_sc=True,
                                         needs_layout_passes=False),
)
def _gather_kernel(table_hbm, idx_hbm, out_hbm,
                   i0, i1, q0, q1, r0, r1, t0, t1,
                   si0, si1, sg0, sg1, so0, so1):
    idx_v = [i0, i1]
    idq_v = [q0, q1]
    rows_v = [r0, r1]
    out_v = [t0, t1]
    isem = [si0, si1]
    gsem = [sg0, sg1]
    osem = [so0, so1]

    wid = lax.axis_index("s") * NC + lax.axis_index("c")
    iota16 = lax.iota(jnp.int32, 16)

    def idx_src(t):
        lt = t // 4
        ls = (t % 4) * HALF
        return idx_hbm.at[pl.ds((lt * NW + wid) * 1024 + ls * 128, ROWS)]

    def out_dst(t):
        return out_hbm.at[pl.ds(t * HALF, HALF), :, pl.ds(wid * 128, 128)]

    pltpu.async_copy(idx_src(0), idx_v[0], isem[0])

    @pl.loop(0, N_STEPS // 2)
    def _pair(jj):
        for p in range(2):
            t = jj * 2 + p
            pltpu.make_async_copy(idx_src(t), idx_v[p], isem[p]).wait()

            @pl.loop(0, ROWS // 16)
            def _prep(c):
                raw = idx_v[p][pl.ds(c * 16, 16)]
                idq_v[p][pl.ds(c * 16, 16)] = raw >> 2

            @pl.when(jj > 0)
            def _():
                pltpu.make_async_copy(out_v[p], out_dst(t - 2),
                                      osem[p]).wait()

            for i in range(HALF):
                pltpu.async_copy(
                    table_hbm.at[idq_v[p].at[pl.ds(i * 128, 128)]],
                    rows_v[p].at[pl.ds(i * 128, 128)],
                    gsem[p])
            if p == 0:
                pltpu.async_copy(idx_src(t + 1), idx_v[1], isem[1])
            else:
                @pl.when(jj < N_STEPS // 2 - 1)
                def _():
                    pltpu.async_copy(idx_src(t + 1), idx_v[0], isem[0])
            for i in range(HALF):
                pltpu.make_async_copy(
                    table_hbm.at[idq_v[p].at[pl.ds(0, 128)]],
                    rows_v[p].at[pl.ds(0, 128)],
                    gsem[p]).wait()

            for lh in range(HALF):
                for jb in range(8):
                    ridx = lh * 128 + jb * 16 + iota16
                    sub = (idx_v[p][pl.ds(lh * 128 + jb * 16, 16)] & 3) * 32

                    @pl.loop(0, EMBED // 8)
                    def _eq(eq):
                        e0 = eq * 8
                        for k in range(8):
                            vals = plsc.load_gather(
                                rows_v[p], [ridx, sub + (e0 + k)])
                            out_v[p][lh, e0 + k,
                                     pl.ds(jb * 16, 16)] = vals

            pltpu.async_copy(out_v[p], out_dst(t), osem[p])

    for t in range(N_STEPS - 2, N_STEPS):
        pltpu.make_async_copy(out_v[t % 2], out_dst(t), osem[t % 2]).wait()


def kernel(indices, table):
    idx_phys = (indices.T.reshape(LT, 8, NW, 128)
                .transpose(0, 2, 1, 3).reshape(BATCH * SEQ))
    tab4 = table.reshape(VOCAB // 4, 4 * EMBED)
    out3 = _gather_kernel(tab4, idx_phys)
    return out3.transpose(2, 0, 1)


# speedup vs baseline: 1.3618x; 1.3618x over previous
import functools

import jax
import jax.numpy as jnp
from jax import lax
from jax.experimental import pallas as pl
from jax.experimental.pallas import tpu as pltpu
from jax.experimental.pallas import tpu_sc as plsc

VOCAB = 1000000
EMBED = 32
BATCH = 4096
SEQ = 200

NC = 2
NS = 16
NW = NC * NS
B_TOTAL = BATCH * SEQ
PER_W = B_TOTAL // NW
CHUNK = 1600
N_CHUNKS = PER_W // CHUNK
N_PAIRS = N_CHUNKS // 2
N_SUB = 4


@functools.partial(
    pl.kernel,
    out_type=jax.ShapeDtypeStruct((B_TOTAL, EMBED), jnp.float32),
    mesh=plsc.VectorSubcoreMesh(
        core_axis_name="c", subcore_axis_name="s",
        num_cores=NC, num_subcores=NS),
    scratch_types=[
        pltpu.VMEM((CHUNK,), jnp.int32),
        pltpu.VMEM((CHUNK,), jnp.int32),
        pltpu.VMEM((CHUNK, EMBED), jnp.float32),
        pltpu.VMEM((CHUNK, EMBED), jnp.float32),
        pltpu.SemaphoreType.DMA,
        pltpu.SemaphoreType.DMA,
        pltpu.SemaphoreType.DMA,
        pltpu.SemaphoreType.DMA,
        pltpu.SemaphoreType.DMA,
        pltpu.SemaphoreType.DMA,
    ],
    compiler_params=pltpu.CompilerParams(use_tc_tiling_on_sc=False),
)
def _gather_kernel(table_hbm, idx_hbm, out_hbm,
                   i0, i1, r0, r1, si0, si1, sg0, sg1, so0, so1):
    idx_v = [i0, i1]
    rows_v = [r0, r1]
    isem = [si0, si1]
    gsem = [sg0, sg1]
    osem = [so0, so1]

    wid = lax.axis_index("s") * NC + lax.axis_index("c")
    base = wid * PER_W

    def idx_start(c, p):
        pltpu.async_copy(idx_hbm.at[pl.ds(base + c * CHUNK, CHUNK)],
                         idx_v[p], isem[p])

    idx_start(0, 0)

    @pl.loop(0, N_PAIRS)
    def _pair(jj):
        for p in range(2):
            c = jj * 2 + p
            off = base + c * CHUNK
            pltpu.make_async_copy(
                idx_hbm.at[pl.ds(off, CHUNK)], idx_v[p], isem[p]).wait()

            @pl.when(jj > 0)
            def _():
                pltpu.make_async_copy(
                    rows_v[p],
                    out_hbm.at[pl.ds(off - 2 * CHUNK, CHUNK)],
                    osem[p]).wait()

            sub = CHUNK // N_SUB
            for q in range(N_SUB):
                pltpu.async_copy(
                    table_hbm.at[idx_v[p].at[pl.ds(q * sub, sub)]],
                    rows_v[p].at[pl.ds(q * sub, sub)],
                    gsem[p])

            if p == 0:
                idx_start(c + 1, 1)
            else:
                @pl.when(jj < N_PAIRS - 1)
                def _():
                    idx_start(c + 1, 0)

            for q in range(N_SUB):
                pltpu.make_async_copy(
                    table_hbm.at[idx_v[p].at[pl.ds(0, CHUNK // N_SUB)]],
                    rows_v[p].at[pl.ds(0, CHUNK // N_SUB)],
                    gsem[p]).wait()
            pltpu.async_copy(rows_v[p], out_hbm.at[pl.ds(off, CHUNK)],
                             osem[p])

    for p in range(2):
        c = N_CHUNKS - 2 + p
        pltpu.make_async_copy(
            rows_v[p], out_hbm.at[pl.ds(base + c * CHUNK, CHUNK)],
            osem[p]).wait()


def kernel(indices, table):
    flat = indices.reshape(B_TOTAL)
    out = _gather_kernel(table, flat)
    return out.reshape(BATCH, SEQ, EMBED)
